# Initial kernel scaffold; baseline (speedup 1.0000x reference)
import jax
import jax.numpy as jnp
from jax.experimental import pallas as pl


def kernel(x, edge_index1, edge_index2, W1, b1, W2, b2):
    raise NotImplementedError("write your pallas kernel here")


# speedup vs baseline: 5.9184x; 5.9184x over previous
import functools

import jax
import jax.numpy as jnp
from jax import lax
from jax.experimental import pallas as pl
from jax.experimental.pallas import tpu as pltpu
from jax.experimental.pallas import tpu_sc as plsc

NC = 2
NS = 16
CHUNK = 128
HD = 16

_mesh = plsc.VectorSubcoreMesh(core_axis_name="c", subcore_axis_name="s")


def _degree_kernel(n_pad, d, ch_per_tile, ib):
    zr = n_pad // NS

    @functools.partial(
        pl.kernel,
        mesh=_mesh,
        out_type=jax.ShapeDtypeStruct((NC, n_pad, d), jnp.float32),
        scratch_types=[
            pltpu.VMEM((2, ib, CHUNK), jnp.int32),
            pltpu.VMEM((2, CHUNK, d), jnp.float32),
            pltpu.VMEM_SHARED((n_pad, d), jnp.float32),
        ],
    )
    def deg(idx_hbm, ones_hbm, zeros_hbm, out_hbm, idx_v, ones_v, acc):
        c = lax.axis_index("c")
        s = lax.axis_index("s")
        wid = c * NS + s
        pltpu.sync_copy(ones_hbm, ones_v)
        pltpu.sync_copy(zeros_hbm, acc.at[pl.ds(s * zr, zr)])
        plsc.subcore_barrier()

        @pl.loop(0, ch_per_tile, step=ib)
        def _(j0):
            for h in range(2):
                pltpu.sync_copy(
                    idx_hbm.at[h].at[pl.ds(wid * ch_per_tile + j0, ib)],
                    idx_v.at[h])

            @pl.loop(0, ib)
            def _(j):
                for h in range(2):
                    pltpu.sync_copy(ones_v.at[h], acc.at[idx_v.at[h].at[j]],
                                    add=True)

        plsc.subcore_barrier()
        pltpu.sync_copy(acc.at[pl.ds(s * zr, zr)],
                        out_hbm.at[c].at[pl.ds(s * zr, zr)])

    return deg


def _gather_scatter_kernel(n_pad, d, ch_per_tile, nv):
    zr = n_pad // NS
    ib = 16
    assert ch_per_tile % ib == 0

    @functools.partial(
        pl.kernel,
        mesh=_mesh,
        out_type=jax.ShapeDtypeStruct((NC, n_pad, d), jnp.float32),
        scratch_types=[
            pltpu.VMEM((ib, CHUNK), jnp.int32),
            pltpu.VMEM((ib, CHUNK), jnp.int32),
            pltpu.VMEM((CHUNK, d), jnp.float32),
            pltpu.VMEM((CHUNK, d), jnp.float32),
            pltpu.VMEM_SHARED((n_pad, d), jnp.float32),
            pltpu.SemaphoreType.DMA,
            pltpu.SemaphoreType.DMA,
        ],
    )
    def gs(y_hbm, srcv_hbm, dstv_hbm, zeros_hbm, out_hbm,
           src_v, dst_v, rows0, rows1, acc, sem0, sem1):
        c = lax.axis_index("c")
        s = lax.axis_index("s")
        wid = c * NS + s
        pltpu.sync_copy(zeros_hbm, acc.at[pl.ds(s * zr, zr)])
        plsc.subcore_barrier()

        @pl.loop(0, ch_per_tile, step=ib)
        def _(j0):
            pltpu.sync_copy(srcv_hbm.at[pl.ds(wid * ch_per_tile + j0, ib)], src_v)
            pltpu.sync_copy(dstv_hbm.at[pl.ds(wid * ch_per_tile + j0, ib)], dst_v)

            @pl.loop(0, ib, step=2)
            def _(j):
                g0 = pltpu.async_copy(y_hbm.at[src_v.at[j]], rows0, sem0)
                g1 = pltpu.async_copy(y_hbm.at[src_v.at[j + 1]], rows1, sem1)
                g0.wait()
                pltpu.sync_copy(rows0, acc.at[dst_v.at[j]], add=True)
                g1.wait()
                pltpu.sync_copy(rows1, acc.at[dst_v.at[j + 1]], add=True)

        plsc.subcore_barrier()
        pltpu.sync_copy(acc.at[pl.ds(s * zr, zr)],
                        out_hbm.at[c].at[pl.ds(s * zr, zr)])

    return gs


def _scale_rows_kernel(n_pad, d, block):
    def body(x_ref, h_ref, o_ref):
        norm = lax.rsqrt(jnp.maximum(h_ref[:, 0:1], 1.0))
        o_ref[...] = x_ref[...] * norm

    return pl.pallas_call(
        body,
        grid=(n_pad // block,),
        in_specs=[
            pl.BlockSpec((block, d), lambda i: (i, 0)),
            pl.BlockSpec((block, HD), lambda i: (i, 0)),
        ],
        out_specs=pl.BlockSpec((block, d), lambda i: (i, 0)),
        out_shape=jax.ShapeDtypeStruct((n_pad, d), jnp.float32),
    )


def _linear_kernel(n_pad, d_in, d_out, block, with_src_norm):
    def body(p0_ref, p1_ref, hd_ref, hs_ref, w_ref, b_ref, o_ref):
        agg = p0_ref[...] + p1_ref[...]
        nd = lax.rsqrt(jnp.maximum(hd_ref[:, 0:1], 1.0))
        h = jnp.dot(agg * nd, w_ref[...],
                    preferred_element_type=jnp.float32,
                    precision=lax.Precision.HIGHEST)
        h = jnp.maximum(h + b_ref[...], 0.0)
        if with_src_norm:
            h = h * lax.rsqrt(jnp.maximum(hs_ref[:, 0:1], 1.0))
        o_ref[...] = h

    return pl.pallas_call(
        body,
        grid=(n_pad // block,),
        in_specs=[
            pl.BlockSpec((block, d_in), lambda i: (i, 0)),
            pl.BlockSpec((block, d_in), lambda i: (i, 0)),
            pl.BlockSpec((block, HD), lambda i: (i, 0)),
            pl.BlockSpec((block, HD), lambda i: (i, 0)),
            pl.BlockSpec((d_in, d_out), lambda i: (0, 0)),
            pl.BlockSpec((1, d_out), lambda i: (0, 0)),
        ],
        out_specs=pl.BlockSpec((block, d_out), lambda i: (i, 0)),
        out_shape=jax.ShapeDtypeStruct((n_pad, d_out), jnp.float32),
    )


def kernel(x, edge_index1, edge_index2, W1, b1, W2, b2):
    n, d = x.shape
    e = edge_index1.shape[1]

    n_pad = ((n + 16 + 127) // 128) * 128
    ch_per_tile = -(-e // (NC * NS * CHUNK))
    ch_per_tile = ((ch_per_tile + 15) // 16) * 16
    e_pad = NC * NS * ch_per_tile * CHUNK
    nv = e_pad // CHUNK
    ib_deg = 8
    assert ch_per_tile % ib_deg == 0

    def pad_idx(idx):
        p = jnp.full((e_pad - e,), n, dtype=jnp.int32)
        return jnp.concatenate([idx.astype(jnp.int32), p]).reshape(nv, CHUNK)

    src1 = pad_idx(edge_index1[0])
    dst1 = pad_idx(edge_index1[1])
    src2 = pad_idx(edge_index2[0])
    dst2 = pad_idx(edge_index2[1])

    x_pad = jnp.zeros((n_pad, d), jnp.float32).at[:n].set(x)
    zr = n_pad // NS
    zeros_d = jnp.zeros((zr, d), jnp.float32)
    col = jnp.arange(d, dtype=jnp.int32)[None, None, :]
    hsel = jnp.arange(2, dtype=jnp.int32)[:, None, None]
    ones2 = jnp.where((col >= 64 * hsel) & (col < 64 * hsel + 64), 1.0, 0.0
                      ).astype(jnp.float32) * jnp.ones((2, CHUNK, d), jnp.float32)

    degk = _degree_kernel(n_pad, d, ch_per_tile, ib_deg)
    dp1 = degk(jnp.stack([src1, dst1]), ones2, zeros_d)
    dp2 = degk(jnp.stack([src2, dst2]), ones2, zeros_d)
    d1 = dp1[0] + dp1[1]
    d2 = dp2[0] + dp2[1]
    deg = jnp.stack([d1[:, 0:HD], d1[:, 64:64 + HD],
                     d2[:, 0:HD], d2[:, 64:64 + HD]])

    block = n_pad // 4
    y1 = _scale_rows_kernel(n_pad, d, block)(x_pad, deg[0])

    gs = _gather_scatter_kernel(n_pad, d, ch_per_tile, nv)
    parts1 = gs(y1, src1, dst1, zeros_d)

    lin1 = _linear_kernel(n_pad, d, W1.shape[1], block, with_src_norm=True)
    y2 = lin1(parts1[0], parts1[1], deg[1], deg[2], W1, b1.reshape(1, -1))

    parts2 = gs(y2, src2, dst2, zeros_d)

    lin2 = _linear_kernel(n_pad, W1.shape[1], W2.shape[1], block, with_src_norm=False)
    out = lin2(parts2[0], parts2[1], deg[3], deg[3], W2, b2.reshape(1, -1))

    return out[:n]


# speedup vs baseline: 6.1024x; 1.0311x over previous
import functools

import jax
import jax.numpy as jnp
from jax import lax
from jax.experimental import pallas as pl
from jax.experimental.pallas import tpu as pltpu
from jax.experimental.pallas import tpu_sc as plsc

NC = 2
NS = 16
CHUNK = 128
HD = 16

_mesh = plsc.VectorSubcoreMesh(core_axis_name="c", subcore_axis_name="s")


def _degree_kernel(n_pad, d, ch_per_tile, ib):
    zr = n_pad // NS

    @functools.partial(
        pl.kernel,
        mesh=_mesh,
        out_type=jax.ShapeDtypeStruct((NC, n_pad, d), jnp.float32),
        scratch_types=[
            pltpu.VMEM((2, ib, CHUNK), jnp.int32),
            pltpu.VMEM((2, CHUNK, d), jnp.float32),
            pltpu.VMEM_SHARED((n_pad, d), jnp.float32),
        ],
    )
    def deg(idx_hbm, ones_hbm, zeros_hbm, out_hbm, idx_v, ones_v, acc):
        c = lax.axis_index("c")
        s = lax.axis_index("s")
        wid = c * NS + s
        pltpu.sync_copy(ones_hbm, ones_v)
        pltpu.sync_copy(zeros_hbm, acc.at[pl.ds(s * zr, zr)])
        plsc.subcore_barrier()

        @pl.loop(0, ch_per_tile, step=ib)
        def _(j0):
            for h in range(2):
                pltpu.sync_copy(
                    idx_hbm.at[h].at[pl.ds(wid * ch_per_tile + j0, ib)],
                    idx_v.at[h])

            @pl.loop(0, ib)
            def _(j):
                for h in range(2):
                    pltpu.sync_copy(ones_v.at[h], acc.at[idx_v.at[h].at[j]],
                                    add=True)

        plsc.subcore_barrier()
        pltpu.sync_copy(acc.at[pl.ds(s * zr, zr)],
                        out_hbm.at[c].at[pl.ds(s * zr, zr)])

    return deg


CH = 64
NB = 4


def _gather_scatter_kernel(n_pad, d, nch_tile):
    zr = n_pad // NS
    ib = 8
    assert nch_tile % ib == 0 and ib % NB == 0

    @functools.partial(
        pl.kernel,
        mesh=_mesh,
        out_type=jax.ShapeDtypeStruct((NC, n_pad, d), jnp.float32),
        scratch_types=[
            pltpu.VMEM((ib, CH), jnp.int32),
            pltpu.VMEM((ib, CH), jnp.int32),
            pltpu.VMEM((NB, CH, d), jnp.float32),
            pltpu.VMEM_SHARED((n_pad, d), jnp.float32),
        ] + [pltpu.SemaphoreType.DMA] * (2 * NB),
    )
    def gs(y_hbm, srcv_hbm, dstv_hbm, zeros_hbm, out_hbm,
           src_v, dst_v, rows, acc, *sems):
        sg = sems[:NB]
        ss = sems[NB:]
        c = lax.axis_index("c")
        s = lax.axis_index("s")
        wid = c * NS + s
        base = wid * nch_tile
        pltpu.sync_copy(zeros_hbm, acc.at[pl.ds(s * zr, zr)])
        plsc.subcore_barrier()

        @pl.loop(0, nch_tile, step=ib)
        def _(j0):
            pltpu.sync_copy(srcv_hbm.at[pl.ds(base + j0, ib)], src_v)
            pltpu.sync_copy(dstv_hbm.at[pl.ds(base + j0, ib)], dst_v)

            gh = [pltpu.async_copy(y_hbm.at[src_v.at[b]], rows.at[b], sg[b])
                  for b in range(NB)]
            sh = [None] * NB
            for k in range(ib):
                b = k % NB
                gh[b].wait()
                sh[b] = pltpu.async_copy(rows.at[b], acc.at[dst_v.at[k]],
                                         ss[b], add=True)
                nk = k + NB
                if nk < ib:
                    sh[b].wait()
                    gh[b] = pltpu.async_copy(y_hbm.at[src_v.at[nk]],
                                             rows.at[b], sg[b])
            for b in range(NB):
                sh[b].wait()

        plsc.subcore_barrier()
        pltpu.sync_copy(acc.at[pl.ds(s * zr, zr)],
                        out_hbm.at[c].at[pl.ds(s * zr, zr)])

    return gs


def _scale_rows_kernel(n_pad, d, block):
    def body(x_ref, h_ref, o_ref):
        norm = lax.rsqrt(jnp.maximum(h_ref[:, 0:1], 1.0))
        o_ref[...] = x_ref[...] * norm

    return pl.pallas_call(
        body,
        grid=(n_pad // block,),
        in_specs=[
            pl.BlockSpec((block, d), lambda i: (i, 0)),
            pl.BlockSpec((block, HD), lambda i: (i, 0)),
        ],
        out_specs=pl.BlockSpec((block, d), lambda i: (i, 0)),
        out_shape=jax.ShapeDtypeStruct((n_pad, d), jnp.float32),
    )


def _linear_kernel(n_pad, d_in, d_out, block, with_src_norm):
    def body(p0_ref, p1_ref, hd_ref, hs_ref, w_ref, b_ref, o_ref):
        agg = p0_ref[...] + p1_ref[...]
        nd = lax.rsqrt(jnp.maximum(hd_ref[:, 0:1], 1.0))
        h = jnp.dot(agg * nd, w_ref[...],
                    preferred_element_type=jnp.float32,
                    precision=lax.Precision.HIGHEST)
        h = jnp.maximum(h + b_ref[...], 0.0)
        if with_src_norm:
            h = h * lax.rsqrt(jnp.maximum(hs_ref[:, 0:1], 1.0))
        o_ref[...] = h

    return pl.pallas_call(
        body,
        grid=(n_pad // block,),
        in_specs=[
            pl.BlockSpec((block, d_in), lambda i: (i, 0)),
            pl.BlockSpec((block, d_in), lambda i: (i, 0)),
            pl.BlockSpec((block, HD), lambda i: (i, 0)),
            pl.BlockSpec((block, HD), lambda i: (i, 0)),
            pl.BlockSpec((d_in, d_out), lambda i: (0, 0)),
            pl.BlockSpec((1, d_out), lambda i: (0, 0)),
        ],
        out_specs=pl.BlockSpec((block, d_out), lambda i: (i, 0)),
        out_shape=jax.ShapeDtypeStruct((n_pad, d_out), jnp.float32),
    )


def kernel(x, edge_index1, edge_index2, W1, b1, W2, b2):
    n, d = x.shape
    e = edge_index1.shape[1]

    n_pad = ((n + 16 + 127) // 128) * 128
    ch_per_tile = -(-e // (NC * NS * CHUNK))
    ch_per_tile = ((ch_per_tile + 15) // 16) * 16
    e_pad = NC * NS * ch_per_tile * CHUNK
    nv = e_pad // CHUNK
    ib_deg = 8
    assert ch_per_tile % ib_deg == 0

    def pad_idx(idx):
        p = jnp.full((e_pad - e,), n, dtype=jnp.int32)
        return jnp.concatenate([idx.astype(jnp.int32), p]).reshape(nv, CHUNK)

    src1 = pad_idx(edge_index1[0])
    dst1 = pad_idx(edge_index1[1])
    src2 = pad_idx(edge_index2[0])
    dst2 = pad_idx(edge_index2[1])

    x_pad = jnp.zeros((n_pad, d), jnp.float32).at[:n].set(x)
    zr = n_pad // NS
    zeros_d = jnp.zeros((zr, d), jnp.float32)
    col = jnp.arange(d, dtype=jnp.int32)[None, None, :]
    hsel = jnp.arange(2, dtype=jnp.int32)[:, None, None]
    ones2 = jnp.where((col >= 64 * hsel) & (col < 64 * hsel + 64), 1.0, 0.0
                      ).astype(jnp.float32) * jnp.ones((2, CHUNK, d), jnp.float32)

    degk = _degree_kernel(n_pad, d, ch_per_tile, ib_deg)
    dp1 = degk(jnp.stack([src1, dst1]), ones2, zeros_d)
    dp2 = degk(jnp.stack([src2, dst2]), ones2, zeros_d)
    d1 = dp1[0] + dp1[1]
    d2 = dp2[0] + dp2[1]
    deg = jnp.stack([d1[:, 0:HD], d1[:, 64:64 + HD],
                     d2[:, 0:HD], d2[:, 64:64 + HD]])

    block = n_pad // 4
    y1 = _scale_rows_kernel(n_pad, d, block)(x_pad, deg[0])

    nv2 = e_pad // CH
    nch_tile = nv2 // (NC * NS)
    gs = _gather_scatter_kernel(n_pad, d, nch_tile)
    s1, t1, s2, t2 = (a.reshape(nv2, CH) for a in (src1, dst1, src2, dst2))
    parts1 = gs(y1, s1, t1, zeros_d)

    lin1 = _linear_kernel(n_pad, d, W1.shape[1], block, with_src_norm=True)
    y2 = lin1(parts1[0], parts1[1], deg[1], deg[2], W1, b1.reshape(1, -1))

    parts2 = gs(y2, s2, t2, zeros_d)

    lin2 = _linear_kernel(n_pad, W1.shape[1], W2.shape[1], block, with_src_norm=False)
    out = lin2(parts2[0], parts2[1], deg[3], deg[3], W2, b2.reshape(1, -1))

    return out[:n]
